# baseline (speedup 1.0000x reference)
import jax
import jax.numpy as jnp
from jax.experimental import pallas as pl
from jax.experimental.pallas import tpu as pltpu

_NUM_PQ = 4
_NUM_CODES = 1024
_EMBED_DIM = 384
_PQ_DIM = _EMBED_DIM // _NUM_PQ
_B = 16
_HW = 24 * 24
_ROWS = _B * _HW


def _vq_block(z_ref, cb_ref, zq_ref, prob_ref, loss_ref):
    zt = z_ref[0, 0]
    cb = cb_ref[0]
    zb = zt.T

    zsq = jnp.sum(zb * zb, axis=1, keepdims=True)
    csq = jnp.sum(cb * cb, axis=1, keepdims=True).T
    cross = jax.lax.dot_general(
        zb, cb, (((1,), (1,)), ((), ())),
        preferred_element_type=jnp.float32)
    dmat = (zsq + csq) - 2.0 * cross

    dmin = jnp.min(dmat, axis=1, keepdims=True)
    t = dmat - dmin

    e = jnp.exp(t * -2.0)
    s = jnp.sum(e, axis=1, keepdims=True)
    prob_ref[0] = e * (1.0 / s)

    iota_f = jax.lax.broadcasted_iota(
        jnp.int32, dmat.shape, 1).astype(jnp.float32)
    key = t * jnp.float32(2.0 ** 34) + iota_f
    kmin = jnp.min(key, axis=1, keepdims=True)

    oh = jnp.where(key == kmin, 1.0, 0.0).astype(jnp.bfloat16)
    zq = jax.lax.dot_general(
        oh, cb.astype(jnp.bfloat16), (((1,), (0,)), ((), ())),
        preferred_element_type=jnp.float32)

    zq_ref[0, 0] = (zb + (zq - zb)).T

    loss_ref[0, 0, 0, 0] = jnp.sum(dmin)


@jax.jit
def kernel(z, codebooks):
    B, C, H, W = z.shape
    z4 = z.reshape(B, _NUM_PQ, _PQ_DIM, _HW)

    zq4, prob3, loss = pl.pallas_call(
        _vq_block,
        grid=(_NUM_PQ, _B),
        in_specs=[
            pl.BlockSpec((1, 1, _PQ_DIM, _HW), lambda g, b: (b, g, 0, 0)),
            pl.BlockSpec((1, _NUM_CODES, _PQ_DIM), lambda g, b: (g, 0, 0)),
        ],
        out_specs=[
            pl.BlockSpec((1, 1, _PQ_DIM, _HW), lambda g, b: (b, g, 0, 0)),
            pl.BlockSpec((1, _HW, _NUM_CODES), lambda g, b: (b, 0, g)),
            pl.BlockSpec((1, 1, 1, 1), lambda g, b: (g, b, 0, 0),
                         memory_space=pltpu.MemorySpace.SMEM),
        ],
        out_shape=[
            jax.ShapeDtypeStruct((B, _NUM_PQ, _PQ_DIM, _HW), jnp.float32),
            jax.ShapeDtypeStruct((B, _HW, _NUM_PQ * _NUM_CODES), jnp.float32),
            jax.ShapeDtypeStruct((_NUM_PQ, _B, 1, 1), jnp.float32),
        ],
        compiler_params=pltpu.CompilerParams(
            dimension_semantics=("parallel", "parallel")),
    )(z4, codebooks)

    z_q = zq4.reshape(B, C, H, W)
    prob = prob3.reshape(_ROWS, _NUM_PQ * _NUM_CODES)
    vq_loss = jnp.sum(loss) * (1.25 / (_NUM_PQ * _ROWS * _PQ_DIM))
    return z_q, vq_loss, prob


# speedup vs baseline: 1.0274x; 1.0274x over previous
import jax
import jax.numpy as jnp
from jax.experimental import pallas as pl
from jax.experimental.pallas import tpu as pltpu

_NUM_PQ = 4
_NUM_CODES = 1024
_EMBED_DIM = 384
_PQ_DIM = _EMBED_DIM // _NUM_PQ
_B = 16
_BB = 2
_HW = 24 * 24
_R = _BB * _HW
_ROWS = _B * _HW


def _vq_block(z_ref, cb_ref, iota_ref, zq_ref, prob_ref, loss_ref):
    zt = z_ref[:, 0]
    cb = cb_ref[0]
    zb = jnp.transpose(zt, (0, 2, 1)).reshape(_R, _PQ_DIM)

    zsq = jnp.sum(zb * zb, axis=1, keepdims=True)
    csq = jnp.sum(cb * cb, axis=1, keepdims=True).T
    cross = jax.lax.dot_general(
        zb, cb, (((1,), (1,)), ((), ())),
        preferred_element_type=jnp.float32)
    dmat = (zsq + csq) - 2.0 * cross

    dmin = jnp.min(dmat, axis=1, keepdims=True)
    t = dmat - dmin

    e = jnp.exp(t * -2.0)
    s = jnp.sum(e, axis=1, keepdims=True)
    p = e * (1.0 / s)
    prob_ref[...] = p.reshape(_BB, _HW, _NUM_CODES)

    key = t * jnp.float32(2.0 ** 34) + iota_ref[...]
    kmin = jnp.min(key, axis=1, keepdims=True)

    oh = jnp.where(key == kmin, 1.0, 0.0).astype(jnp.bfloat16)
    zq = jax.lax.dot_general(
        oh, cb.astype(jnp.bfloat16), (((1,), (0,)), ((), ())),
        preferred_element_type=jnp.float32)

    st = (zb + (zq - zb)).reshape(_BB, _HW, _PQ_DIM)
    zq_ref[:, 0] = jnp.transpose(st, (0, 2, 1))

    loss_ref[0, 0, 0, 0] = jnp.sum(dmin)


@jax.jit
def kernel(z, codebooks):
    B, C, H, W = z.shape
    z4 = z.reshape(B, _NUM_PQ, _PQ_DIM, _HW)
    iota_row = jax.lax.broadcasted_iota(
        jnp.float32, (1, _NUM_CODES), 1)

    zq4, prob3, loss = pl.pallas_call(
        _vq_block,
        grid=(_NUM_PQ, _B // _BB),
        in_specs=[
            pl.BlockSpec((_BB, 1, _PQ_DIM, _HW), lambda g, b: (b, g, 0, 0)),
            pl.BlockSpec((1, _NUM_CODES, _PQ_DIM), lambda g, b: (g, 0, 0)),
            pl.BlockSpec((1, _NUM_CODES), lambda g, b: (0, 0)),
        ],
        out_specs=[
            pl.BlockSpec((_BB, 1, _PQ_DIM, _HW), lambda g, b: (b, g, 0, 0)),
            pl.BlockSpec((_BB, _HW, _NUM_CODES), lambda g, b: (b, 0, g)),
            pl.BlockSpec((1, 1, 1, 1), lambda g, b: (g, b, 0, 0),
                         memory_space=pltpu.MemorySpace.SMEM),
        ],
        out_shape=[
            jax.ShapeDtypeStruct((B, _NUM_PQ, _PQ_DIM, _HW), jnp.float32),
            jax.ShapeDtypeStruct((B, _HW, _NUM_PQ * _NUM_CODES), jnp.float32),
            jax.ShapeDtypeStruct((_NUM_PQ, _B // _BB, 1, 1), jnp.float32),
        ],
        compiler_params=pltpu.CompilerParams(
            dimension_semantics=("parallel", "parallel")),
    )(z4, codebooks, iota_row)

    z_q = zq4.reshape(B, C, H, W)
    prob = prob3.reshape(_ROWS, _NUM_PQ * _NUM_CODES)
    vq_loss = jnp.sum(loss) * (1.25 / (_NUM_PQ * _ROWS * _PQ_DIM))
    return z_q, vq_loss, prob


# speedup vs baseline: 1.0494x; 1.0214x over previous
import jax
import jax.numpy as jnp
from jax.experimental import pallas as pl
from jax.experimental.pallas import tpu as pltpu

_NUM_PQ = 4
_NUM_CODES = 1024
_EMBED_DIM = 384
_PQ_DIM = _EMBED_DIM // _NUM_PQ
_B = 16
_BB = 2
_HW = 24 * 24
_R = _BB * _HW
_ROWS = _B * _HW


def _vq_block(z_ref, cb_ref, iota_ref, zq_ref, prob_ref, loss_ref):
    zt = z_ref[:, 0]
    cb = cb_ref[0]
    zb = jnp.transpose(zt, (0, 2, 1)).reshape(_R, _PQ_DIM)

    zsq = jnp.sum(zb * zb, axis=1, keepdims=True)
    csq = jnp.sum(cb * cb, axis=1, keepdims=True).T
    cross = jax.lax.dot_general(
        zb, cb, (((1,), (1,)), ((), ())),
        preferred_element_type=jnp.float32)
    dmat = (zsq + csq) - 2.0 * cross

    dmin = jnp.min(dmat, axis=1, keepdims=True)
    t = dmat - dmin

    e = jnp.exp2(t * jnp.float32(-2.885390081777927))
    s = jnp.sum(e, axis=1, keepdims=True)
    p = e * (1.0 / s)
    prob_ref[...] = p.reshape(_BB, _HW, _NUM_CODES)

    key = t * jnp.float32(2.0 ** 34) + iota_ref[...]
    kmin = jnp.min(key, axis=1, keepdims=True)

    oh = jnp.where(key == kmin, 1.0, 0.0).astype(jnp.bfloat16)
    zq = jax.lax.dot_general(
        oh, cb.astype(jnp.bfloat16), (((1,), (0,)), ((), ())),
        preferred_element_type=jnp.float32)

    st = (zb + (zq - zb)).reshape(_BB, _HW, _PQ_DIM)
    zq_ref[:, 0] = jnp.transpose(st, (0, 2, 1))

    loss_ref[0, 0, 0, 0] = jnp.sum(dmin)


@jax.jit
def kernel(z, codebooks):
    B, C, H, W = z.shape
    z4 = z.reshape(B, _NUM_PQ, _PQ_DIM, _HW)
    iota_row = jax.lax.broadcasted_iota(
        jnp.float32, (1, _NUM_CODES), 1)

    zq4, prob3, loss = pl.pallas_call(
        _vq_block,
        grid=(_NUM_PQ, _B // _BB),
        in_specs=[
            pl.BlockSpec((_BB, 1, _PQ_DIM, _HW), lambda g, b: (b, g, 0, 0)),
            pl.BlockSpec((1, _NUM_CODES, _PQ_DIM), lambda g, b: (g, 0, 0)),
            pl.BlockSpec((1, _NUM_CODES), lambda g, b: (0, 0)),
        ],
        out_specs=[
            pl.BlockSpec((_BB, 1, _PQ_DIM, _HW), lambda g, b: (b, g, 0, 0)),
            pl.BlockSpec((_BB, _HW, _NUM_CODES), lambda g, b: (b, 0, g)),
            pl.BlockSpec((1, 1, 1, 1), lambda g, b: (g, b, 0, 0),
                         memory_space=pltpu.MemorySpace.SMEM),
        ],
        out_shape=[
            jax.ShapeDtypeStruct((B, _NUM_PQ, _PQ_DIM, _HW), jnp.float32),
            jax.ShapeDtypeStruct((B, _HW, _NUM_PQ * _NUM_CODES), jnp.float32),
            jax.ShapeDtypeStruct((_NUM_PQ, _B // _BB, 1, 1), jnp.float32),
        ],
        compiler_params=pltpu.CompilerParams(
            dimension_semantics=("parallel", "parallel")),
    )(z4, codebooks, iota_row)

    z_q = zq4.reshape(B, C, H, W)
    prob = prob3.reshape(_ROWS, _NUM_PQ * _NUM_CODES)
    vq_loss = jnp.sum(loss) * (1.25 / (_NUM_PQ * _ROWS * _PQ_DIM))
    return z_q, vq_loss, prob
